# baseline (speedup 1.0000x reference)
import functools

import jax
import jax.numpy as jnp
from jax import lax
from jax.experimental import pallas as pl
from jax.experimental.pallas import tpu as pltpu
from jax.experimental.pallas import tpu_sc as plsc

_NC = 2
_NS = 16


def _sc_gather(x, idx_segs):
    nw = idx_segs[0].shape[0]
    d = x.shape[1]
    segs = [(a.shape[1], a.shape[2]) for a in idx_segs]
    per_w = sum(nc * ch for nc, ch in segs)
    total = nw * per_w
    mesh = plsc.VectorSubcoreMesh(core_axis_name="c", subcore_axis_name="s")

    scratch = [pltpu.VMEM((nc, ch), jnp.int32) for nc, ch in segs]
    scratch += [pltpu.VMEM((ch, d), x.dtype) for _, ch in segs]
    scratch += [pltpu.SemaphoreType.DMA]

    @functools.partial(
        pl.kernel,
        out_type=jax.ShapeDtypeStruct((total, d), x.dtype),
        mesh=mesh,
        scratch_types=scratch,
    )
    def k(x_hbm, *refs):
        nseg = len(segs)
        idx_hbms = refs[:nseg]
        out_hbm = refs[nseg]
        idx_vs = refs[nseg + 1:2 * nseg + 1]
        buf_vs = refs[2 * nseg + 1:3 * nseg + 1]
        sem = refs[3 * nseg + 1]
        wid = lax.axis_index("s") * _NC + lax.axis_index("c")
        base = wid * per_w
        off = 0
        for i, (nc, ch) in enumerate(segs):
            pltpu.sync_copy(idx_hbms[i].at[wid], idx_vs[i])

            @pl.loop(0, nc)
            def _(j, i=i, ch=ch, off=off):
                pltpu.async_copy(x_hbm.at[idx_vs[i].at[j]], buf_vs[i], sem).wait()
                pltpu.sync_copy(buf_vs[i],
                                out_hbm.at[pl.ds(base + off + j * ch, ch)])

            off += nc * ch

    return k(x, *idx_segs)


def _sc_scatter_residual(h_parts, dst_segs, init_src):
    n, d = init_src.shape
    dh = d // _NC
    ns = dst_segs[0].shape[0]
    segs = [(a.shape[1], a.shape[2]) for a in dst_segs]
    nseg = len(segs)
    per_s = sum(nc * ch for nc, ch in segs)
    nparts = len(h_parts)
    part_rows = h_parts[0].shape[0]
    sub_per_part = part_rows // per_s
    rows_main = (n // ns) // 8 * 8
    tail_base = ns * rows_main
    tail_rows = n - tail_base
    mesh = plsc.VectorSubcoreMesh(core_axis_name="c", subcore_axis_name="s")

    scratch = [pltpu.VMEM((nc, ch), jnp.int32) for nc, ch in segs]
    scratch += [pltpu.VMEM((ch, dh), init_src.dtype) for _, ch in segs]
    scratch += [pltpu.VMEM_SHARED((n, dh), init_src.dtype)]

    @functools.partial(
        pl.kernel,
        out_type=jax.ShapeDtypeStruct((n, d), init_src.dtype),
        mesh=mesh,
        scratch_types=scratch,
    )
    def k(*refs):
        h_refs = refs[:nparts]
        dst_hbms = refs[nparts:nparts + nseg]
        x_hbm = refs[nparts + nseg]
        out_hbm = refs[nparts + nseg + 1]
        idx_vs = refs[nparts + nseg + 2:nparts + nseg + 2 + nseg]
        buf_vs = refs[nparts + nseg + 2 + nseg:nparts + nseg + 2 + 2 * nseg]
        acc_sh = refs[-1]
        c = lax.axis_index("c")
        s = lax.axis_index("s")
        col0 = c * dh
        r0 = s * rows_main
        pltpu.sync_copy(
            x_hbm.at[pl.ds(r0, rows_main), pl.ds(col0, dh)],
            acc_sh.at[pl.ds(r0, rows_main)],
        )
        if tail_rows:
            @pl.when(s == ns - 1)
            def _():
                pltpu.sync_copy(
                    x_hbm.at[pl.ds(tail_base, tail_rows), pl.ds(col0, dh)],
                    acc_sh.at[pl.ds(tail_base, tail_rows)],
                )
        for i in range(nseg):
            pltpu.sync_copy(dst_hbms[i].at[s], idx_vs[i])
        plsc.subcore_barrier()

        sub_local = lax.rem(s, sub_per_part)
        for kp in range(nparts):
            @pl.when(s // sub_per_part == kp)
            def _(kp=kp):
                off = 0
                for i, (nc, ch) in enumerate(segs):
                    @pl.loop(0, nc)
                    def _(j, i=i, ch=ch, off=off):
                        rbase = sub_local * per_s + off + j * ch
                        pltpu.sync_copy(
                            h_refs[kp].at[pl.ds(rbase, ch), pl.ds(col0, dh)],
                            buf_vs[i])
                        pltpu.sync_copy(buf_vs[i], acc_sh.at[idx_vs[i].at[j]],
                                        add=True)

                    off += nc * ch

        plsc.subcore_barrier()
        pltpu.sync_copy(
            acc_sh.at[pl.ds(r0, rows_main)],
            out_hbm.at[pl.ds(r0, rows_main), pl.ds(col0, dh)],
        )
        if tail_rows:
            @pl.when(s == ns - 1)
            def _():
                pltpu.sync_copy(
                    acc_sh.at[pl.ds(tail_base, tail_rows)],
                    out_hbm.at[pl.ds(tail_base, tail_rows), pl.ds(col0, dh)],
                )

    return k(*h_parts, *dst_segs, init_src)


def _tc_lstm(seq_all, steps, w_ih, w_hh, bias, blk):
    lp, d = seq_all.shape
    g = w_ih.shape[0]
    p = lp // steps
    nblk = p // blk
    prec = lax.Precision.DEFAULT
    dn = (((1,), (1,)), ((), ()))

    def body(*refs):
        s_refs = refs[:steps]
        wih_ref, whh_ref, b_ref, out_ref = refs[steps:]
        wih = wih_ref[...]
        whh = whh_ref[...]
        b = b_ref[...]
        half = blk // 2

        def step(t, h, c, lo):
            st = s_refs[t][pl.ds(lo, half), :]
            gates = lax.dot_general(st, wih, dn, precision=prec,
                                    preferred_element_type=jnp.float32) + b
            if h is not None:
                gates = gates + lax.dot_general(h, whh, dn, precision=prec,
                                                preferred_element_type=jnp.float32)
            gi = jax.nn.sigmoid(gates[:, 0 * d:1 * d])
            gg = jnp.tanh(gates[:, 2 * d:3 * d])
            go = jax.nn.sigmoid(gates[:, 3 * d:4 * d])
            if c is None:
                c = gi * gg
            else:
                gf = jax.nn.sigmoid(gates[:, 1 * d:2 * d])
                c = gf * c + gi * gg
            return go * jnp.tanh(c), c

        ha = ca = hb = cb = None
        for t in range(steps):
            ha, ca = step(t, ha, ca, 0)
            hb, cb = step(t, hb, cb, half)
        out_ref[pl.ds(0, half), :] = ha
        out_ref[pl.ds(half, half), :] = hb

    seq_specs = [
        pl.BlockSpec((blk, d), lambda i, t=t: (t * nblk + i, 0))
        for t in range(steps)
    ]
    return pl.pallas_call(
        body,
        grid=(nblk,),
        in_specs=seq_specs + [
            pl.BlockSpec((g, d), lambda i: (0, 0)),
            pl.BlockSpec((g, d), lambda i: (0, 0)),
            pl.BlockSpec((1, g), lambda i: (0, 0)),
        ],
        out_specs=pl.BlockSpec((blk, d), lambda i: (i, 0)),
        out_shape=jax.ShapeDtypeStruct((p, d), jnp.float32),
    )(*([seq_all] * steps), w_ih, w_hh, bias)


def _tc_bn_relu(y, gamma, beta):
    n, d = y.shape

    def body(y_ref, g_ref, b_ref, o_ref):
        v = y_ref[...]
        mean = jnp.mean(v, axis=0, keepdims=True)
        cent = v - mean
        var = jnp.mean(cent * cent, axis=0, keepdims=True)
        scaled = cent * lax.rsqrt(var + 1e-5) * g_ref[...] + b_ref[...]
        o_ref[...] = jnp.maximum(scaled, 0.0)

    return pl.pallas_call(
        body,
        out_shape=jax.ShapeDtypeStruct((n, d), y.dtype),
    )(y, gamma.reshape(1, d), beta.reshape(1, d))


def kernel(x, paths, W_ih, W_hh, b_ih, b_hh, gamma, beta):
    n, d = x.shape
    p, l = paths.shape
    paths = paths.astype(jnp.int32)
    bias = (b_ih + b_hh).reshape(1, 4 * d).astype(jnp.float32)

    nw = _NC * _NS
    chunk = 80
    n_slices = 4
    ps = p // n_slices
    per_w = (ps * l) // nw

    def _split(flat2d, width):
        n_main = width // chunk
        tail = width - n_main * chunk
        rows = flat2d.shape[0]
        segs = [flat2d[:, :n_main * chunk].reshape(rows, n_main, chunk)]
        if tail:
            segs.append(flat2d[:, n_main * chunk:].reshape(rows, 1, tail))
        return segs

    h_parts = []
    for k in range(n_slices):
        pk = paths[k * ps:(k + 1) * ps]
        seq_k = _sc_gather(x, _split(pk.T.reshape(nw, per_w), per_w))
        h_parts.append(_tc_lstm(seq_k, l, W_ih, W_hh, bias, blk=2000))

    dst = paths[:, l - 1]
    half = p // 2
    per_s = half // _NS
    segs_a = _split(dst[:half].reshape(_NS, per_s), per_s)
    segs_b = _split(dst[half:].reshape(_NS, per_s), per_s)
    y_a = _sc_scatter_residual(h_parts[:n_slices // 2], segs_a, x)
    y = _sc_scatter_residual(h_parts[n_slices // 2:], segs_b, y_a)

    return _tc_bn_relu(y, gamma, beta)


# speedup vs baseline: 1.0156x; 1.0156x over previous
import functools

import jax
import jax.numpy as jnp
from jax import lax
from jax.experimental import pallas as pl
from jax.experimental.pallas import tpu as pltpu
from jax.experimental.pallas import tpu_sc as plsc

_NC = 2
_NS = 16


def _sc_gather(x, idx_segs):
    nw = idx_segs[0].shape[0]
    d = x.shape[1]
    segs = [(a.shape[1], a.shape[2]) for a in idx_segs]
    per_w = sum(nc * ch for nc, ch in segs)
    total = nw * per_w
    mesh = plsc.VectorSubcoreMesh(core_axis_name="c", subcore_axis_name="s")

    nbuf = 4
    ch0 = segs[0][1]
    scratch = [pltpu.VMEM((nc, ch), jnp.int32) for nc, ch in segs]
    scratch += [pltpu.VMEM((ch0, d), x.dtype) for _ in range(nbuf)]
    scratch += [pltpu.SemaphoreType.DMA for _ in range(2 * nbuf)]
    scratch += [pltpu.VMEM((segs[i][1], d), x.dtype) for i in range(1, len(segs))]
    scratch += [pltpu.SemaphoreType.DMA]

    @functools.partial(
        pl.kernel,
        out_type=jax.ShapeDtypeStruct((total, d), x.dtype),
        mesh=mesh,
        scratch_types=scratch,
    )
    def k(x_hbm, *refs):
        nseg = len(segs)
        idx_hbms = refs[:nseg]
        out_hbm = refs[nseg]
        r = list(refs[nseg + 1:])
        idx_vs = r[:nseg]
        bufs = r[nseg:nseg + nbuf]
        gs = r[nseg + nbuf:nseg + 2 * nbuf]
        ws = r[nseg + 2 * nbuf:nseg + 3 * nbuf]
        tail_bufs = r[nseg + 3 * nbuf:nseg + 3 * nbuf + nseg - 1]
        sem = r[-1]
        wid = lax.axis_index("s") * _NC + lax.axis_index("c")
        base = wid * per_w
        for i in range(nseg):
            pltpu.sync_copy(idx_hbms[i].at[wid], idx_vs[i])

        nc0 = segs[0][0]
        assert nc0 % nbuf == 0
        nit = nc0 // nbuf

        def g_copy(j, b):
            return pltpu.make_async_copy(x_hbm.at[idx_vs[0].at[j]], bufs[b],
                                         gs[b])

        def w_copy(j, b):
            return pltpu.make_async_copy(
                bufs[b], out_hbm.at[pl.ds(base + j * ch0, ch0)], ws[b])

        g_copy(0, 0).start()

        @pl.loop(0, nit)
        def _(it):
            for b in range(nbuf):
                j = it * nbuf + b
                g_copy(j, b).wait()
                w_copy(j, b).start()
                bn = (b + 1) % nbuf
                if b < nbuf - 1:
                    @pl.when(it > 0)
                    def _(j=j, bn=bn):
                        w_copy(j + 1 - nbuf, bn).wait()
                    g_copy(j + 1, bn).start()
                else:
                    @pl.when(it < nit - 1)
                    def _(j=j, bn=bn):
                        w_copy(j + 1 - nbuf, bn).wait()
                        g_copy(j + 1, bn).start()

        for j in range(nc0 - nbuf, nc0):
            w_copy(j, j % nbuf).wait()

        off = nc0 * ch0
        for i in range(1, nseg):
            nc, ch = segs[i]

            @pl.loop(0, nc)
            def _(j, i=i, ch=ch, off=off):
                pltpu.async_copy(x_hbm.at[idx_vs[i].at[j]], tail_bufs[i - 1],
                                 sem).wait()
                pltpu.sync_copy(tail_bufs[i - 1],
                                out_hbm.at[pl.ds(base + off + j * ch, ch)])

            off += nc * ch

    return k(x, *idx_segs)


def _sc_scatter_residual(h_parts, dst_segs, init_src):
    n, d = init_src.shape
    dh = d // _NC
    ns = dst_segs[0].shape[0]
    segs = [(a.shape[1], a.shape[2]) for a in dst_segs]
    nseg = len(segs)
    per_s = sum(nc * ch for nc, ch in segs)
    nparts = len(h_parts)
    part_rows = h_parts[0].shape[0]
    sub_per_part = part_rows // per_s
    rows_main = (n // ns) // 8 * 8
    tail_base = ns * rows_main
    tail_rows = n - tail_base
    mesh = plsc.VectorSubcoreMesh(core_axis_name="c", subcore_axis_name="s")

    scratch = [pltpu.VMEM((nc, ch), jnp.int32) for nc, ch in segs]
    scratch += [pltpu.VMEM((ch, dh), init_src.dtype) for _, ch in segs]
    scratch += [pltpu.VMEM_SHARED((n, dh), init_src.dtype)]

    @functools.partial(
        pl.kernel,
        out_type=jax.ShapeDtypeStruct((n, d), init_src.dtype),
        mesh=mesh,
        scratch_types=scratch,
    )
    def k(*refs):
        h_refs = refs[:nparts]
        dst_hbms = refs[nparts:nparts + nseg]
        x_hbm = refs[nparts + nseg]
        out_hbm = refs[nparts + nseg + 1]
        idx_vs = refs[nparts + nseg + 2:nparts + nseg + 2 + nseg]
        buf_vs = refs[nparts + nseg + 2 + nseg:nparts + nseg + 2 + 2 * nseg]
        acc_sh = refs[-1]
        c = lax.axis_index("c")
        s = lax.axis_index("s")
        col0 = c * dh
        r0 = s * rows_main
        pltpu.sync_copy(
            x_hbm.at[pl.ds(r0, rows_main), pl.ds(col0, dh)],
            acc_sh.at[pl.ds(r0, rows_main)],
        )
        if tail_rows:
            @pl.when(s == ns - 1)
            def _():
                pltpu.sync_copy(
                    x_hbm.at[pl.ds(tail_base, tail_rows), pl.ds(col0, dh)],
                    acc_sh.at[pl.ds(tail_base, tail_rows)],
                )
        for i in range(nseg):
            pltpu.sync_copy(dst_hbms[i].at[s], idx_vs[i])
        plsc.subcore_barrier()

        sub_local = lax.rem(s, sub_per_part)
        for kp in range(nparts):
            @pl.when(s // sub_per_part == kp)
            def _(kp=kp):
                off = 0
                for i, (nc, ch) in enumerate(segs):
                    @pl.loop(0, nc)
                    def _(j, i=i, ch=ch, off=off):
                        rbase = sub_local * per_s + off + j * ch
                        pltpu.sync_copy(
                            h_refs[kp].at[pl.ds(rbase, ch), pl.ds(col0, dh)],
                            buf_vs[i])
                        pltpu.sync_copy(buf_vs[i], acc_sh.at[idx_vs[i].at[j]],
                                        add=True)

                    off += nc * ch

        plsc.subcore_barrier()
        pltpu.sync_copy(
            acc_sh.at[pl.ds(r0, rows_main)],
            out_hbm.at[pl.ds(r0, rows_main), pl.ds(col0, dh)],
        )
        if tail_rows:
            @pl.when(s == ns - 1)
            def _():
                pltpu.sync_copy(
                    acc_sh.at[pl.ds(tail_base, tail_rows)],
                    out_hbm.at[pl.ds(tail_base, tail_rows), pl.ds(col0, dh)],
                )

    return k(*h_parts, *dst_segs, init_src)


def _tc_lstm(seq_all, steps, w_ih, w_hh, bias, blk):
    lp, d = seq_all.shape
    g = w_ih.shape[0]
    p = lp // steps
    nblk = p // blk
    prec = lax.Precision.DEFAULT
    dn = (((1,), (1,)), ((), ()))

    def body(*refs):
        s_refs = refs[:steps]
        wih_ref, whh_ref, b_ref, out_ref = refs[steps:]
        wih = wih_ref[...]
        whh = whh_ref[...]
        b = b_ref[...]
        half = blk // 2

        def step(t, h, c, lo):
            st = s_refs[t][pl.ds(lo, half), :]
            gates = lax.dot_general(st, wih, dn, precision=prec,
                                    preferred_element_type=jnp.float32) + b
            if h is not None:
                gates = gates + lax.dot_general(h, whh, dn, precision=prec,
                                                preferred_element_type=jnp.float32)
            gi = jax.nn.sigmoid(gates[:, 0 * d:1 * d])
            gg = jnp.tanh(gates[:, 2 * d:3 * d])
            go = jax.nn.sigmoid(gates[:, 3 * d:4 * d])
            if c is None:
                c = gi * gg
            else:
                gf = jax.nn.sigmoid(gates[:, 1 * d:2 * d])
                c = gf * c + gi * gg
            return go * jnp.tanh(c), c

        ha = ca = hb = cb = None
        for t in range(steps):
            ha, ca = step(t, ha, ca, 0)
            hb, cb = step(t, hb, cb, half)
        out_ref[pl.ds(0, half), :] = ha
        out_ref[pl.ds(half, half), :] = hb

    seq_specs = [
        pl.BlockSpec((blk, d), lambda i, t=t: (t * nblk + i, 0))
        for t in range(steps)
    ]
    return pl.pallas_call(
        body,
        grid=(nblk,),
        in_specs=seq_specs + [
            pl.BlockSpec((g, d), lambda i: (0, 0)),
            pl.BlockSpec((g, d), lambda i: (0, 0)),
            pl.BlockSpec((1, g), lambda i: (0, 0)),
        ],
        out_specs=pl.BlockSpec((blk, d), lambda i: (i, 0)),
        out_shape=jax.ShapeDtypeStruct((p, d), jnp.float32),
    )(*([seq_all] * steps), w_ih, w_hh, bias)


def _tc_bn_relu(y, gamma, beta):
    n, d = y.shape

    def body(y_ref, g_ref, b_ref, o_ref):
        v = y_ref[...]
        mean = jnp.mean(v, axis=0, keepdims=True)
        cent = v - mean
        var = jnp.mean(cent * cent, axis=0, keepdims=True)
        scaled = cent * lax.rsqrt(var + 1e-5) * g_ref[...] + b_ref[...]
        o_ref[...] = jnp.maximum(scaled, 0.0)

    return pl.pallas_call(
        body,
        out_shape=jax.ShapeDtypeStruct((n, d), y.dtype),
    )(y, gamma.reshape(1, d), beta.reshape(1, d))


def kernel(x, paths, W_ih, W_hh, b_ih, b_hh, gamma, beta):
    n, d = x.shape
    p, l = paths.shape
    paths = paths.astype(jnp.int32)
    bias = (b_ih + b_hh).reshape(1, 4 * d).astype(jnp.float32)

    nw = _NC * _NS
    chunk = 80
    n_slices = 4
    ps = p // n_slices
    per_w = (ps * l) // nw

    def _split(flat2d, width, main_mult=1):
        n_chunks = width // chunk
        n_main = n_chunks - n_chunks % main_mult
        tail = width - n_chunks * chunk
        rows = flat2d.shape[0]
        segs = [flat2d[:, :n_main * chunk].reshape(rows, n_main, chunk)]
        if n_chunks > n_main:
            segs.append(
                flat2d[:, n_main * chunk:n_chunks * chunk]
                .reshape(rows, n_chunks - n_main, chunk))
        if tail:
            segs.append(flat2d[:, n_chunks * chunk:].reshape(rows, 1, tail))
        return segs

    h_parts = []
    for k in range(n_slices):
        pk = paths[k * ps:(k + 1) * ps]
        seq_k = _sc_gather(x, _split(pk.T.reshape(nw, per_w), per_w,
                                     main_mult=4))
        h_parts.append(_tc_lstm(seq_k, l, W_ih, W_hh, bias, blk=2000))

    dst = paths[:, l - 1]
    half = p // 2
    per_s = half // _NS
    segs_a = _split(dst[:half].reshape(_NS, per_s), per_s)
    segs_b = _split(dst[half:].reshape(_NS, per_s), per_s)
    y_a = _sc_scatter_residual(h_parts[:n_slices // 2], segs_a, x)
    y = _sc_scatter_residual(h_parts[n_slices // 2:], segs_b, y_a)

    return _tc_bn_relu(y, gamma, beta)


# speedup vs baseline: 1.0346x; 1.0187x over previous
import functools

import jax
import jax.numpy as jnp
from jax import lax
from jax.experimental import pallas as pl
from jax.experimental.pallas import tpu as pltpu
from jax.experimental.pallas import tpu_sc as plsc

_NC = 2
_NS = 16


def _sc_gather(x, idx_segs):
    nw = idx_segs[0].shape[0]
    d = x.shape[1]
    segs = [(a.shape[1], a.shape[2]) for a in idx_segs]
    per_w = sum(nc * ch for nc, ch in segs)
    total = nw * per_w
    mesh = plsc.VectorSubcoreMesh(core_axis_name="c", subcore_axis_name="s")

    nbuf = 4
    ch0 = segs[0][1]
    scratch = [pltpu.VMEM((nc, ch), jnp.int32) for nc, ch in segs]
    scratch += [pltpu.VMEM((ch0, d), x.dtype) for _ in range(nbuf)]
    scratch += [pltpu.SemaphoreType.DMA for _ in range(2 * nbuf)]
    scratch += [pltpu.VMEM((segs[i][1], d), x.dtype) for i in range(1, len(segs))]
    scratch += [pltpu.SemaphoreType.DMA]

    @functools.partial(
        pl.kernel,
        out_type=jax.ShapeDtypeStruct((total, d), x.dtype),
        mesh=mesh,
        scratch_types=scratch,
    )
    def k(x_hbm, *refs):
        nseg = len(segs)
        idx_hbms = refs[:nseg]
        out_hbm = refs[nseg]
        r = list(refs[nseg + 1:])
        idx_vs = r[:nseg]
        bufs = r[nseg:nseg + nbuf]
        gs = r[nseg + nbuf:nseg + 2 * nbuf]
        ws = r[nseg + 2 * nbuf:nseg + 3 * nbuf]
        tail_bufs = r[nseg + 3 * nbuf:nseg + 3 * nbuf + nseg - 1]
        sem = r[-1]
        wid = lax.axis_index("s") * _NC + lax.axis_index("c")
        base = wid * per_w
        for i in range(nseg):
            pltpu.sync_copy(idx_hbms[i].at[wid], idx_vs[i])

        nc0 = segs[0][0]
        assert nc0 % nbuf == 0
        nit = nc0 // nbuf

        def g_copy(j, b):
            return pltpu.make_async_copy(x_hbm.at[idx_vs[0].at[j]], bufs[b],
                                         gs[b])

        def w_copy(j, b):
            return pltpu.make_async_copy(
                bufs[b], out_hbm.at[pl.ds(base + j * ch0, ch0)], ws[b])

        g_copy(0, 0).start()

        @pl.loop(0, nit)
        def _(it):
            for b in range(nbuf):
                j = it * nbuf + b
                g_copy(j, b).wait()
                w_copy(j, b).start()
                bn = (b + 1) % nbuf
                if b < nbuf - 1:
                    @pl.when(it > 0)
                    def _(j=j, bn=bn):
                        w_copy(j + 1 - nbuf, bn).wait()
                    g_copy(j + 1, bn).start()
                else:
                    @pl.when(it < nit - 1)
                    def _(j=j, bn=bn):
                        w_copy(j + 1 - nbuf, bn).wait()
                        g_copy(j + 1, bn).start()

        for j in range(nc0 - nbuf, nc0):
            w_copy(j, j % nbuf).wait()

        off = nc0 * ch0
        for i in range(1, nseg):
            nc, ch = segs[i]

            @pl.loop(0, nc)
            def _(j, i=i, ch=ch, off=off):
                pltpu.async_copy(x_hbm.at[idx_vs[i].at[j]], tail_bufs[i - 1],
                                 sem).wait()
                pltpu.sync_copy(tail_bufs[i - 1],
                                out_hbm.at[pl.ds(base + off + j * ch, ch)])

            off += nc * ch

    return k(x, *idx_segs)


def _sc_scatter_residual(h_parts, dst_segs, init_src):
    n, d = init_src.shape
    dh = d // _NC
    ns = dst_segs[0].shape[0]
    segs = [(a.shape[1], a.shape[2]) for a in dst_segs]
    nseg = len(segs)
    per_s = sum(nc * ch for nc, ch in segs)
    nparts = len(h_parts)
    part_rows = h_parts[0].shape[0]
    sub_per_part = part_rows // per_s
    rows_main = (n // ns) // 8 * 8
    tail_base = ns * rows_main
    tail_rows = n - tail_base
    mesh = plsc.VectorSubcoreMesh(core_axis_name="c", subcore_axis_name="s")

    scratch = [pltpu.VMEM((nc, ch), jnp.int32) for nc, ch in segs]
    scratch += [pltpu.VMEM((ch, dh), init_src.dtype) for _, ch in segs]
    scratch += [pltpu.VMEM((segs[0][1], dh), init_src.dtype),
                pltpu.SemaphoreType.DMA, pltpu.SemaphoreType.DMA]
    scratch += [pltpu.VMEM_SHARED((n, dh), init_src.dtype)]

    @functools.partial(
        pl.kernel,
        out_type=jax.ShapeDtypeStruct((n, d), init_src.dtype),
        mesh=mesh,
        scratch_types=scratch,
    )
    def k(*refs):
        h_refs = refs[:nparts]
        dst_hbms = refs[nparts:nparts + nseg]
        x_hbm = refs[nparts + nseg]
        out_hbm = refs[nparts + nseg + 1]
        idx_vs = refs[nparts + nseg + 2:nparts + nseg + 2 + nseg]
        buf_vs = refs[nparts + nseg + 2 + nseg:nparts + nseg + 2 + 2 * nseg]
        buf2_v = refs[-4]
        rsems = refs[-3:-1]
        acc_sh = refs[-1]
        c = lax.axis_index("c")
        s = lax.axis_index("s")
        col0 = c * dh
        r0 = s * rows_main
        pltpu.sync_copy(
            x_hbm.at[pl.ds(r0, rows_main), pl.ds(col0, dh)],
            acc_sh.at[pl.ds(r0, rows_main)],
        )
        if tail_rows:
            @pl.when(s == ns - 1)
            def _():
                pltpu.sync_copy(
                    x_hbm.at[pl.ds(tail_base, tail_rows), pl.ds(col0, dh)],
                    acc_sh.at[pl.ds(tail_base, tail_rows)],
                )
        for i in range(nseg):
            pltpu.sync_copy(dst_hbms[i].at[s], idx_vs[i])
        plsc.subcore_barrier()

        sub_local = lax.rem(s, sub_per_part)
        nc0, ch0 = segs[0]
        assert nc0 % 2 == 0
        bufs2 = (buf_vs[0], buf2_v)
        for kp in range(nparts):
            @pl.when(s // sub_per_part == kp)
            def _(kp=kp):
                def rd(j, b):
                    rbase = sub_local * per_s + j * ch0
                    return pltpu.make_async_copy(
                        h_refs[kp].at[pl.ds(rbase, ch0), pl.ds(col0, dh)],
                        bufs2[b], rsems[b])

                rd(0, 0).start()

                @pl.loop(0, nc0 // 2)
                def _(it):
                    j0 = it * 2
                    rd(j0, 0).wait()
                    rd(j0 + 1, 1).start()
                    pltpu.sync_copy(bufs2[0], acc_sh.at[idx_vs[0].at[j0]],
                                    add=True)
                    rd(j0 + 1, 1).wait()

                    @pl.when(it < nc0 // 2 - 1)
                    def _():
                        rd(j0 + 2, 0).start()
                    pltpu.sync_copy(bufs2[1], acc_sh.at[idx_vs[0].at[j0 + 1]],
                                    add=True)

                off = nc0 * ch0
                for i, (nc, ch) in enumerate(segs):
                    if i == 0:
                        continue

                    @pl.loop(0, nc)
                    def _(j, i=i, ch=ch, off=off):
                        rbase = sub_local * per_s + off + j * ch
                        pltpu.sync_copy(
                            h_refs[kp].at[pl.ds(rbase, ch), pl.ds(col0, dh)],
                            buf_vs[i])
                        pltpu.sync_copy(buf_vs[i], acc_sh.at[idx_vs[i].at[j]],
                                        add=True)

                    off += nc * ch

        plsc.subcore_barrier()
        pltpu.sync_copy(
            acc_sh.at[pl.ds(r0, rows_main)],
            out_hbm.at[pl.ds(r0, rows_main), pl.ds(col0, dh)],
        )
        if tail_rows:
            @pl.when(s == ns - 1)
            def _():
                pltpu.sync_copy(
                    acc_sh.at[pl.ds(tail_base, tail_rows)],
                    out_hbm.at[pl.ds(tail_base, tail_rows), pl.ds(col0, dh)],
                )

    return k(*h_parts, *dst_segs, init_src)


def _tc_lstm(seq_all, steps, w_ih, w_hh, bias, blk):
    lp, d = seq_all.shape
    g = w_ih.shape[0]
    p = lp // steps
    nblk = p // blk
    prec = lax.Precision.DEFAULT
    dn = (((1,), (1,)), ((), ()))

    def body(*refs):
        s_refs = refs[:steps]
        wih_ref, whh_ref, b_ref, out_ref = refs[steps:]
        wih = wih_ref[...]
        whh = whh_ref[...]
        b = b_ref[...]
        half = blk // 2

        def step(t, h, c, lo):
            st = s_refs[t][pl.ds(lo, half), :]
            gates = lax.dot_general(st, wih, dn, precision=prec,
                                    preferred_element_type=jnp.float32) + b
            if h is not None:
                gates = gates + lax.dot_general(h, whh, dn, precision=prec,
                                                preferred_element_type=jnp.float32)
            gi = jax.nn.sigmoid(gates[:, 0 * d:1 * d])
            gg = jnp.tanh(gates[:, 2 * d:3 * d])
            go = jax.nn.sigmoid(gates[:, 3 * d:4 * d])
            if c is None:
                c = gi * gg
            else:
                gf = jax.nn.sigmoid(gates[:, 1 * d:2 * d])
                c = gf * c + gi * gg
            return go * jnp.tanh(c), c

        ha = ca = hb = cb = None
        for t in range(steps):
            ha, ca = step(t, ha, ca, 0)
            hb, cb = step(t, hb, cb, half)
        out_ref[pl.ds(0, half), :] = ha
        out_ref[pl.ds(half, half), :] = hb

    seq_specs = [
        pl.BlockSpec((blk, d), lambda i, t=t: (t * nblk + i, 0))
        for t in range(steps)
    ]
    return pl.pallas_call(
        body,
        grid=(nblk,),
        in_specs=seq_specs + [
            pl.BlockSpec((g, d), lambda i: (0, 0)),
            pl.BlockSpec((g, d), lambda i: (0, 0)),
            pl.BlockSpec((1, g), lambda i: (0, 0)),
        ],
        out_specs=pl.BlockSpec((blk, d), lambda i: (i, 0)),
        out_shape=jax.ShapeDtypeStruct((p, d), jnp.float32),
    )(*([seq_all] * steps), w_ih, w_hh, bias)


def _tc_bn_relu(y, gamma, beta):
    n, d = y.shape

    def body(y_ref, g_ref, b_ref, o_ref):
        v = y_ref[...]
        mean = jnp.mean(v, axis=0, keepdims=True)
        cent = v - mean
        var = jnp.mean(cent * cent, axis=0, keepdims=True)
        scaled = cent * lax.rsqrt(var + 1e-5) * g_ref[...] + b_ref[...]
        o_ref[...] = jnp.maximum(scaled, 0.0)

    return pl.pallas_call(
        body,
        out_shape=jax.ShapeDtypeStruct((n, d), y.dtype),
    )(y, gamma.reshape(1, d), beta.reshape(1, d))


def kernel(x, paths, W_ih, W_hh, b_ih, b_hh, gamma, beta):
    n, d = x.shape
    p, l = paths.shape
    paths = paths.astype(jnp.int32)
    bias = (b_ih + b_hh).reshape(1, 4 * d).astype(jnp.float32)

    nw = _NC * _NS
    chunk = 80
    n_slices = 4
    ps = p // n_slices
    per_w = (ps * l) // nw

    def _split(flat2d, width, main_mult=1):
        n_chunks = width // chunk
        n_main = n_chunks - n_chunks % main_mult
        tail = width - n_chunks * chunk
        rows = flat2d.shape[0]
        segs = [flat2d[:, :n_main * chunk].reshape(rows, n_main, chunk)]
        if n_chunks > n_main:
            segs.append(
                flat2d[:, n_main * chunk:n_chunks * chunk]
                .reshape(rows, n_chunks - n_main, chunk))
        if tail:
            segs.append(flat2d[:, n_chunks * chunk:].reshape(rows, 1, tail))
        return segs

    h_parts = []
    for k in range(n_slices):
        pk = paths[k * ps:(k + 1) * ps]
        seq_k = _sc_gather(x, _split(pk.T.reshape(nw, per_w), per_w,
                                     main_mult=4))
        h_parts.append(_tc_lstm(seq_k, l, W_ih, W_hh, bias, blk=2000))

    dst = paths[:, l - 1]
    half = p // 2
    per_s = half // _NS
    segs_a = _split(dst[:half].reshape(_NS, per_s), per_s)
    segs_b = _split(dst[half:].reshape(_NS, per_s), per_s)
    y_a = _sc_scatter_residual(h_parts[:n_slices // 2], segs_a, x)
    y = _sc_scatter_residual(h_parts[n_slices // 2:], segs_b, y_a)

    return _tc_bn_relu(y, gamma, beta)
